# baseline (speedup 1.0000x reference)
import functools

import jax
import jax.numpy as jnp
from jax import lax
from jax.experimental import pallas as pl
from jax.experimental.pallas import tpu as pltpu
from jax.experimental.pallas import tpu_sc as plsc

HIDDEN = 128
TZ_SECONDS = 8 * 3600
HOURS = 24
DAYS = 7
NUM_COMBOS = HOURS * DAYS
NC, NS, LANES = 2, 16, 16
NW = NC * NS
SUB = 80
RING = 8
LAG = 4


@functools.cache
def _make_gather(total):
    assert total % (NW * SUB) == 0
    b_per_w = total // NW
    n_sub = b_per_w // SUB
    n_groups = n_sub // RING
    assert n_groups * RING == n_sub and n_groups >= 1

    mesh = plsc.VectorSubcoreMesh(core_axis_name="c", subcore_axis_name="s")

    scratch = [
        pltpu.VMEM((b_per_w,), jnp.int32),
        pltpu.VMEM((RING, SUB), jnp.int32),
        pltpu.VMEM((RING, SUB, HIDDEN), jnp.float32),
        pltpu.VMEM((HOURS, HIDDEN), jnp.float32),
        pltpu.VMEM((DAYS, HIDDEN), jnp.float32),
        pltpu.VMEM((NUM_COMBOS, HIDDEN), jnp.float32),
        pltpu.VMEM_SHARED((NUM_COMBOS, HIDDEN), jnp.float32),
        pltpu.SemaphoreType.DMA,
    ] + [pltpu.SemaphoreType.DMA] * (2 * RING)

    @functools.partial(
        pl.kernel,
        out_type=jax.ShapeDtypeStruct((total, HIDDEN), jnp.float32),
        mesh=mesh,
        scratch_types=scratch,
    )
    def sc_kernel(ts_hbm, hour_hbm, day_hbm, out_hbm,
                  ts_v, idx_v, rows_v, hour_v, day_v, tab_v, tab_sh, tsem, *sems):
        gsem, ssem = sems[:RING], sems[RING:]
        sid = lax.axis_index("s")
        wid = sid * NC + lax.axis_index("c")
        base = wid * b_per_w

        ts_copy = pltpu.make_async_copy(ts_hbm.at[pl.ds(base, b_per_w)], ts_v, tsem)
        ts_copy.start()

        @pl.when(sid == 0)
        def _():
            pltpu.sync_copy(hour_hbm, hour_v)
            pltpu.sync_copy(day_hbm, day_v)

            def build(e, carry):
                hr = lax.rem(e, HOURS)
                d = lax.div(e, HOURS)
                for i in range(HIDDEN // LANES):
                    tab_v[e, pl.ds(i * LANES, LANES)] = (
                        hour_v[hr, pl.ds(i * LANES, LANES)]
                        + day_v[d, pl.ds(i * LANES, LANES)]
                    )
                return carry

            lax.fori_loop(0, NUM_COMBOS, build, 0)
            pltpu.sync_copy(tab_v, tab_sh)

        plsc.subcore_barrier()
        ts_copy.wait()

        def idx_compute(j, b):
            for i in range(SUB // LANES):
                t = ts_v[pl.ds(j * SUB + i * LANES, LANES)]
                e = lax.rem(lax.div(t + TZ_SECONDS, 3600), NUM_COMBOS)
                idx_v[b, pl.ds(i * LANES, LANES)] = e

        def gather_start(b):
            pltpu.async_copy(tab_sh.at[idx_v.at[b]], rows_v.at[b], gsem[b])

        def gather_wait(b):
            pltpu.make_async_copy(tab_sh.at[idx_v.at[b]], rows_v.at[b], gsem[b]).wait()

        def store_start(j, b):
            pltpu.async_copy(rows_v.at[b], out_hbm.at[pl.ds(base + j * SUB, SUB)], ssem[b])

        def store_wait(b):
            pltpu.make_async_copy(out_hbm.at[pl.ds(base, SUB)], rows_v.at[b], ssem[b]).wait()

        def group(g, carry):
            for b in range(RING):
                j = g * RING + b
                jj = j - LAG
                b2 = (b - LAG) % RING

                @pl.when(g >= 1)
                def _(b=b):
                    store_wait(b)

                idx_compute(j, b)
                gather_start(b)

                if b >= LAG:
                    gather_wait(b2)
                    store_start(jj, b2)
                else:
                    @pl.when(g >= 1)
                    def _(jj=jj, b2=b2):
                        gather_wait(b2)
                        store_start(jj, b2)
            return carry

        lax.fori_loop(0, n_groups, group, 0)

        for k in range(LAG):
            jj = n_sub - LAG + k
            gather_wait(jj % RING)
            store_start(jj, jj % RING)
        for b in range(RING):
            store_wait(b)

    return sc_kernel


def kernel(timestamp, hour_table, day_table):
    batch, hist = timestamp.shape
    ts_flat = timestamp.T.reshape(-1)
    out = _make_gather(batch * hist)(ts_flat, hour_table, day_table)
    return out.reshape(hist, batch, HIDDEN).transpose(1, 0, 2)
